# Initial kernel scaffold; baseline (speedup 1.0000x reference)
import functools

import jax
import jax.numpy as jnp
from jax.experimental import pallas as pl
from jax.experimental.pallas import tpu as pltpu

S, D, E, K, H = 2048, 768, 8, 2, 3072
BT = 256
SB = S // BT

_NEG_INF = float("-inf")


def _gating_kernel(x_ref, gw_ref, noise_ref, w_ref):
    l = jax.lax.dot_general(
        x_ref[...], gw_ref[...],
        dimension_numbers=(((1,), (1,)), ((), ())),
        preferred_element_type=jnp.float32,
    )
    ln = l + noise_ref[...]
    iota_e = jax.lax.broadcasted_iota(jnp.int32, (S, E), 1)
    m0 = jnp.max(ln, axis=1, keepdims=True)
    e0 = jnp.min(jnp.where(ln == m0, iota_e, E), axis=1, keepdims=True)
    sel0 = iota_e == e0
    ln1 = jnp.where(sel0, _NEG_INF, ln)
    m1 = jnp.max(ln1, axis=1, keepdims=True)
    e1 = jnp.min(jnp.where(ln1 == m1, iota_e, E), axis=1, keepdims=True)
    sel1 = iota_e == e1
    t = jnp.exp(m1 - m0)
    p0 = 1.0 / (1.0 + t)
    p1 = t / (1.0 + t)
    w_ref[...] = jnp.where(sel0, p0, 0.0) + jnp.where(sel1, p1, 0.0)


def _ffn_kernel(x_ref, w_ref, W1_ref, b1_ref, W2_ref, b2_ref, Wp_ref, bp_ref,
                out_ref, acc_ref):
    e = pl.program_id(0)
    sb = pl.program_id(1)
    xb = x_ref[...].astype(jnp.bfloat16)
    h1 = jax.lax.dot_general(
        xb, W1_ref[0], (((1,), (1,)), ((), ())),
        preferred_element_type=jnp.float32) + b1_ref[...]
    h2 = jax.lax.dot_general(
        xb, W2_ref[0], (((1,), (1,)), ((), ())),
        preferred_element_type=jnp.float32) + b2_ref[...]
    h = h1 * (h2 * jax.nn.sigmoid(h2))
    y = jax.lax.dot_general(
        h.astype(jnp.bfloat16), Wp_ref[0], (((1,), (1,)), ((), ())),
        preferred_element_type=jnp.float32) + bp_ref[...]
    iota_e = jax.lax.broadcasted_iota(jnp.int32, (BT, E), 1)
    wcol = jnp.sum(jnp.where(iota_e == e, w_ref[...], 0.0), axis=1,
                   keepdims=True)
    contrib = y * wcol

    @pl.when(e == 0)
    def _():
        acc_ref[pl.ds(sb * BT, BT), :] = contrib

    @pl.when(e > 0)
    def _():
        acc_ref[pl.ds(sb * BT, BT), :] += contrib

    @pl.when(e == E - 1)
    def _():
        out_ref[...] = acc_ref[pl.ds(sb * BT, BT), :]


@jax.jit
def kernel(x, gate_w, noise_w, W1, b1, W2, b2, Wp, bp):
    x_flat = x.reshape(S, D)
    noise_unit = jax.random.normal(jax.random.key(1), (1, S, E),
                                   dtype=jnp.float32)
    noise = (noise_unit * noise_w).reshape(S, E)

    w = pl.pallas_call(
        _gating_kernel,
        out_shape=jax.ShapeDtypeStruct((S, E), jnp.float32),
    )(x_flat, gate_w, noise)

    W1b = W1.astype(jnp.bfloat16)
    W2b = W2.astype(jnp.bfloat16)
    Wpb = Wp.astype(jnp.bfloat16)

    out = pl.pallas_call(
        _ffn_kernel,
        grid=(E, SB),
        in_specs=[
            pl.BlockSpec((BT, D), lambda e, sb: (sb, 0)),
            pl.BlockSpec((BT, E), lambda e, sb: (sb, 0)),
            pl.BlockSpec((1, H, D), lambda e, sb: (e, 0, 0)),
            pl.BlockSpec((1, H), lambda e, sb: (e, 0)),
            pl.BlockSpec((1, H, D), lambda e, sb: (e, 0, 0)),
            pl.BlockSpec((1, H), lambda e, sb: (e, 0)),
            pl.BlockSpec((1, D, H), lambda e, sb: (e, 0, 0)),
            pl.BlockSpec((1, D), lambda e, sb: (e, 0)),
        ],
        out_specs=pl.BlockSpec((BT, D), lambda e, sb: (sb, 0)),
        out_shape=jax.ShapeDtypeStruct((S, D), jnp.float32),
        scratch_shapes=[pltpu.VMEM((S, D), jnp.float32)],
    )(x_flat, w, W1b, b1, W2b, b2, Wpb, bp)

    return out.reshape(1, S, D)


# speedup vs baseline: 1.0890x; 1.0890x over previous
import functools

import jax
import jax.numpy as jnp
from jax.experimental import pallas as pl
from jax.experimental.pallas import tpu as pltpu

S, D, E, K, H = 2048, 768, 8, 2, 3072
BT = 256
SB = S // BT

_NEG_INF = float("-inf")


def _gating_kernel(x_ref, gw_ref, noise_ref, w_ref):
    l = jax.lax.dot_general(
        x_ref[...], gw_ref[...],
        dimension_numbers=(((1,), (1,)), ((), ())),
        preferred_element_type=jnp.float32,
    )
    ln = l + noise_ref[...]
    iota_e = jax.lax.broadcasted_iota(jnp.int32, (S, E), 1)
    m0 = jnp.max(ln, axis=1, keepdims=True)
    e0 = jnp.min(jnp.where(ln == m0, iota_e, E), axis=1, keepdims=True)
    sel0 = iota_e == e0
    ln1 = jnp.where(sel0, _NEG_INF, ln)
    m1 = jnp.max(ln1, axis=1, keepdims=True)
    e1 = jnp.min(jnp.where(ln1 == m1, iota_e, E), axis=1, keepdims=True)
    sel1 = iota_e == e1
    t = jnp.exp(m1 - m0)
    p0 = 1.0 / (1.0 + t)
    p1 = t / (1.0 + t)
    w_ref[...] = jnp.where(sel0, p0, 0.0) + jnp.where(sel1, p1, 0.0)


def _ffn_kernel(x_ref, w_ref, W1_ref, b1_ref, W2_ref, b2_ref, Wp_ref, bp_ref,
                out_ref, acc_ref):
    e = pl.program_id(0)
    sb = pl.program_id(1)
    xb = x_ref[...].astype(jnp.bfloat16)
    h1 = jax.lax.dot_general(
        xb, W1_ref[0], (((1,), (1,)), ((), ())),
        preferred_element_type=jnp.float32) + b1_ref[0]
    h2 = jax.lax.dot_general(
        xb, W2_ref[0], (((1,), (1,)), ((), ())),
        preferred_element_type=jnp.float32) + b2_ref[0]
    h = h1 * (h2 * jax.nn.sigmoid(h2))
    y = jax.lax.dot_general(
        h.astype(jnp.bfloat16), Wp_ref[0], (((1,), (1,)), ((), ())),
        preferred_element_type=jnp.float32) + bp_ref[0]
    iota_e = jax.lax.broadcasted_iota(jnp.int32, (BT, E), 1)
    wcol = jnp.sum(jnp.where(iota_e == e, w_ref[...], 0.0), axis=1,
                   keepdims=True)
    contrib = y * wcol

    @pl.when(e == 0)
    def _():
        acc_ref[pl.ds(sb * BT, BT), :] = contrib

    @pl.when(e > 0)
    def _():
        acc_ref[pl.ds(sb * BT, BT), :] += contrib

    @pl.when(e == E - 1)
    def _():
        out_ref[...] = acc_ref[pl.ds(sb * BT, BT), :]


@jax.jit
def kernel(x, gate_w, noise_w, W1, b1, W2, b2, Wp, bp):
    x_flat = x.reshape(S, D)
    noise_unit = jax.random.normal(jax.random.key(1), (1, S, E),
                                   dtype=jnp.float32)
    noise = (noise_unit * noise_w).reshape(S, E)

    w = pl.pallas_call(
        _gating_kernel,
        out_shape=jax.ShapeDtypeStruct((S, E), jnp.float32),
    )(x_flat, gate_w, noise)

    W1b = W1.astype(jnp.bfloat16)
    W2b = W2.astype(jnp.bfloat16)
    Wpb = Wp.astype(jnp.bfloat16)
    b1r = b1.reshape(E, 1, H)
    b2r = b2.reshape(E, 1, H)
    bpr = bp.reshape(E, 1, D)

    out = pl.pallas_call(
        _ffn_kernel,
        grid=(E, SB),
        in_specs=[
            pl.BlockSpec((BT, D), lambda e, sb: (sb, 0)),
            pl.BlockSpec((BT, E), lambda e, sb: (sb, 0)),
            pl.BlockSpec((1, H, D), lambda e, sb: (e, 0, 0)),
            pl.BlockSpec((1, 1, H), lambda e, sb: (e, 0, 0)),
            pl.BlockSpec((1, H, D), lambda e, sb: (e, 0, 0)),
            pl.BlockSpec((1, 1, H), lambda e, sb: (e, 0, 0)),
            pl.BlockSpec((1, D, H), lambda e, sb: (e, 0, 0)),
            pl.BlockSpec((1, 1, D), lambda e, sb: (e, 0, 0)),
        ],
        out_specs=pl.BlockSpec((BT, D), lambda e, sb: (sb, 0)),
        out_shape=jax.ShapeDtypeStruct((S, D), jnp.float32),
        scratch_shapes=[pltpu.VMEM((S, D), jnp.float32)],
    )(x_flat, w, W1b, b1r, W2b, b2r, Wpb, bpr)

    return out.reshape(1, S, D)


# speedup vs baseline: 1.6029x; 1.4718x over previous
import functools

import jax
import jax.numpy as jnp
from jax import lax
from jax.experimental import pallas as pl
from jax.experimental.pallas import tpu as pltpu
from jax.experimental.pallas import tpu_sc as plsc

S, D, E, K, H = 2048, 768, 8, 2, 3072
BLK = 256
NBLK = S * K // BLK + E
NPAD = NBLK * BLK
NW = 32
TPW = S // NW

_NEG_INF = float("-inf")


def _routing_kernel(x_ref, gw_ref, noise_ref, w0_ref, w1_ref, pos0_ref,
                    pos1_ref, be_ref, nb_ref):
    logits = jax.lax.dot_general(
        x_ref[...], gw_ref[...],
        dimension_numbers=(((1,), (1,)), ((), ())),
        preferred_element_type=jnp.float32,
    )
    ln = logits + noise_ref[...]
    iota_e = lax.broadcasted_iota(jnp.int32, (S, E), 1)
    m0 = jnp.max(ln, axis=1, keepdims=True)
    e0 = jnp.min(jnp.where(ln == m0, iota_e, E), axis=1, keepdims=True)
    sel0 = iota_e == e0
    ln1 = jnp.where(sel0, _NEG_INF, ln)
    m1 = jnp.max(ln1, axis=1, keepdims=True)
    e1 = jnp.min(jnp.where(ln1 == m1, iota_e, E), axis=1, keepdims=True)
    sel1 = iota_e == e1
    t = jnp.exp(m1 - m0)
    w0_ref[...] = 1.0 / (1.0 + t)
    w1_ref[...] = t / (1.0 + t)

    mask = jnp.where(sel0 | sel1, 1.0, 0.0)
    r_iota = lax.broadcasted_iota(jnp.int32, (S, S), 0)
    c_iota = lax.broadcasted_iota(jnp.int32, (S, S), 1)
    slt = jnp.where(c_iota < r_iota, 1.0, 0.0)
    ranks = jax.lax.dot_general(
        slt, mask, (((1,), (0,)), ((), ())),
        preferred_element_type=jnp.float32)
    counts = jnp.sum(mask, axis=0, keepdims=True)
    ci = counts.astype(jnp.int32)
    pci = ((ci + (BLK - 1)) // BLK) * BLK
    ei = lax.broadcasted_iota(jnp.int32, (E, E), 0)
    ej = lax.broadcasted_iota(jnp.int32, (E, E), 1)
    ltE = jnp.where(ei < ej, 1.0, 0.0)
    po = jax.lax.dot_general(
        pci.astype(jnp.float32), ltE, (((1,), (0,)), ((), ())),
        preferred_element_type=jnp.float32)
    pos0f = jnp.sum(jnp.where(sel0, po + ranks, 0.0), axis=1, keepdims=True)
    pos1f = jnp.sum(jnp.where(sel1, po + ranks, 0.0), axis=1, keepdims=True)
    pos0_ref[...] = pos0f.astype(jnp.int32)
    pos1_ref[...] = pos1f.astype(jnp.int32)

    nb = jnp.sum(pci, axis=1, keepdims=True) // BLK
    nb_ref[...] = nb
    bidx = lax.broadcasted_iota(jnp.int32, (NBLK, 1), 0)
    bclamp = jnp.minimum(bidx, nb - 1)
    start_row = (bclamp * BLK).astype(jnp.float32)
    po_b = jnp.broadcast_to(po, (NBLK, E))
    be = jnp.sum(jnp.where(po_b <= start_row, 1, 0), axis=1,
                 keepdims=True) - 1
    be_ref[...] = be.astype(jnp.int32)


def _sc_scatter_kernel(x_hbm, pos0_hbm, pos1_hbm, w0_hbm, w1_hbm,
                       xg_hbm, wg_hbm, xv, p0v, p1v, wv, sem):
    wid = lax.axis_index("s") * 2 + lax.axis_index("c")
    base = wid * TPW
    pltpu.sync_copy(x_hbm.at[pl.ds(base, TPW)], xv)
    pltpu.sync_copy(pos0_hbm.at[pl.ds(base, TPW)], p0v)
    pltpu.sync_copy(pos1_hbm.at[pl.ds(base, TPW)], p1v)
    pltpu.async_copy(xv, xg_hbm.at[p0v], sem).wait()
    pltpu.async_copy(xv, xg_hbm.at[p1v], sem).wait()
    pltpu.sync_copy(w0_hbm.at[pl.ds(base, TPW)], wv)
    pltpu.async_copy(wv, wg_hbm.at[p0v], sem).wait()
    pltpu.sync_copy(w1_hbm.at[pl.ds(base, TPW)], wv)
    pltpu.async_copy(wv, wg_hbm.at[p1v], sem).wait()


def _ffn_kernel(be_ref, nb_ref, xg_ref, wg_ref, W1_ref, b1_ref, W2_ref,
                b2_ref, Wp_ref, bp_ref, yg_ref):
    b = pl.program_id(0)

    @pl.when(b < nb_ref[0])
    def _():
        xb = xg_ref[...].astype(jnp.bfloat16)
        h1 = jax.lax.dot_general(
            xb, W1_ref[0], (((1,), (1,)), ((), ())),
            preferred_element_type=jnp.float32) + b1_ref[0]
        h2 = jax.lax.dot_general(
            xb, W2_ref[0], (((1,), (1,)), ((), ())),
            preferred_element_type=jnp.float32) + b2_ref[0]
        h = h1 * (h2 * jax.nn.sigmoid(h2))
        y = jax.lax.dot_general(
            h.astype(jnp.bfloat16), Wp_ref[0], (((1,), (1,)), ((), ())),
            preferred_element_type=jnp.float32) + bp_ref[0]
        yg_ref[...] = y * wg_ref[...]


def _sc_combine_kernel(yg_hbm, pos0_hbm, pos1_hbm, out_hbm,
                       p0v, p1v, y0v, y1v, sem0, sem1):
    wid = lax.axis_index("s") * 2 + lax.axis_index("c")
    base = wid * TPW
    pltpu.sync_copy(pos0_hbm.at[pl.ds(base, TPW)], p0v)
    pltpu.sync_copy(pos1_hbm.at[pl.ds(base, TPW)], p1v)
    cp0 = pltpu.async_copy(yg_hbm.at[p0v], y0v, sem0)
    cp1 = pltpu.async_copy(yg_hbm.at[p1v], y1v, sem1)
    cp0.wait()
    cp1.wait()

    def _row(r, carry):
        for c in range(D // 16):
            sl = pl.ds(c * 16, 16)
            y0v[r, sl] = y0v[r, sl] + y1v[r, sl]
        return carry

    lax.fori_loop(0, TPW, _row, 0)
    pltpu.sync_copy(y0v, out_hbm.at[pl.ds(base, TPW)])


@functools.lru_cache(maxsize=None)
def _sc_kernels():
    mesh = plsc.VectorSubcoreMesh(core_axis_name="c", subcore_axis_name="s")
    scatter = pl.kernel(
        _sc_scatter_kernel,
        out_type=[
            jax.ShapeDtypeStruct((NPAD, D), jnp.float32),
            jax.ShapeDtypeStruct((NPAD,), jnp.float32),
        ],
        mesh=mesh,
        scratch_types=[
            pltpu.VMEM((TPW, D), jnp.float32),
            pltpu.VMEM((TPW,), jnp.int32),
            pltpu.VMEM((TPW,), jnp.int32),
            pltpu.VMEM((TPW,), jnp.float32),
            pltpu.SemaphoreType.DMA,
        ],
    )
    combine = pl.kernel(
        _sc_combine_kernel,
        out_type=jax.ShapeDtypeStruct((S, D), jnp.float32),
        mesh=mesh,
        scratch_types=[
            pltpu.VMEM((TPW,), jnp.int32),
            pltpu.VMEM((TPW,), jnp.int32),
            pltpu.VMEM((TPW, D), jnp.float32),
            pltpu.VMEM((TPW, D), jnp.float32),
            pltpu.SemaphoreType.DMA,
            pltpu.SemaphoreType.DMA,
        ],
    )
    return scatter, combine


@jax.jit
def kernel(x, gate_w, noise_w, W1, b1, W2, b2, Wp, bp):
    x_flat = x.reshape(S, D)
    noise_unit = jax.random.normal(jax.random.key(1), (1, S, E),
                                   dtype=jnp.float32)
    noise = (noise_unit * noise_w).reshape(S, E)

    w0, w1, pos0, pos1, be, nb = pl.pallas_call(
        _routing_kernel,
        out_shape=[
            jax.ShapeDtypeStruct((S, 1), jnp.float32),
            jax.ShapeDtypeStruct((S, 1), jnp.float32),
            jax.ShapeDtypeStruct((S, 1), jnp.int32),
            jax.ShapeDtypeStruct((S, 1), jnp.int32),
            jax.ShapeDtypeStruct((NBLK, 1), jnp.int32),
            jax.ShapeDtypeStruct((1, 1), jnp.int32),
        ],
    )(x_flat, gate_w, noise)

    pos0_f = pos0.reshape(S)
    pos1_f = pos1.reshape(S)
    sc_scatter, sc_combine = _sc_kernels()
    xg, wg = sc_scatter(x_flat, pos0_f, pos1_f, w0.reshape(S), w1.reshape(S))

    W1b = W1.astype(jnp.bfloat16)
    W2b = W2.astype(jnp.bfloat16)
    Wpb = Wp.astype(jnp.bfloat16)
    b1r = b1.reshape(E, 1, H)
    b2r = b2.reshape(E, 1, H)
    bpr = bp.reshape(E, 1, D)

    yg = pl.pallas_call(
        _ffn_kernel,
        grid_spec=pltpu.PrefetchScalarGridSpec(
            num_scalar_prefetch=2,
            grid=(NBLK,),
            in_specs=[
                pl.BlockSpec((BLK, D), lambda b, be, nb: (b, 0)),
                pl.BlockSpec((BLK, 1), lambda b, be, nb: (b, 0)),
                pl.BlockSpec((1, H, D), lambda b, be, nb: (be[b], 0, 0)),
                pl.BlockSpec((1, 1, H), lambda b, be, nb: (be[b], 0, 0)),
                pl.BlockSpec((1, H, D), lambda b, be, nb: (be[b], 0, 0)),
                pl.BlockSpec((1, 1, H), lambda b, be, nb: (be[b], 0, 0)),
                pl.BlockSpec((1, D, H), lambda b, be, nb: (be[b], 0, 0)),
                pl.BlockSpec((1, 1, D), lambda b, be, nb: (be[b], 0, 0)),
            ],
            out_specs=pl.BlockSpec((BLK, D), lambda b, be, nb: (b, 0)),
        ),
        out_shape=jax.ShapeDtypeStruct((NPAD, D), jnp.float32),
    )(be.reshape(NBLK), nb.reshape(1), xg, wg.reshape(NPAD, 1), W1b, b1r,
      W2b, b2r, Wpb, bpr)

    out = sc_combine(yg, pos0_f, pos1_f)
    return out.reshape(1, S, D)
